# baseline (speedup 1.0000x reference)
import functools

import jax
import jax.numpy as jnp
from jax import lax
from jax.experimental import pallas as pl
from jax.experimental.pallas import tpu as pltpu
from jax.experimental.pallas import tpu_sc as plsc

NC = 2
NS = 16
NW = NC * NS

BB = 4096 // 128
TT = 200


def _fused_gather(xr, tablew):
    mesh = plsc.VectorSubcoreMesh(core_axis_name="c", subcore_axis_name="s")

    @functools.partial(
        pl.kernel,
        out_type=jax.ShapeDtypeStruct((TT, 8, BB, 8, 128), jnp.float32),
        mesh=mesh,
        scratch_types=[
            pltpu.VMEM((128 * TT,), jnp.int32),
            pltpu.VMEM((2, 128), jnp.int32),
            pltpu.VMEM((2, 128), jnp.int32),
            pltpu.VMEM((2, 128, 128), jnp.float32),
            pltpu.VMEM((2, 8, 8, 128), jnp.float32),
            pltpu.SemaphoreType.DMA((2,)),
            pltpu.SemaphoreType.DMA((2,)),
        ],
        compiler_params=pltpu.CompilerParams(
            use_tc_tiling_on_sc=True, needs_layout_passes=False
        ),
    )
    def k(x_hbm, tw_hbm, out_hbm, xv, widx, hoff, rows, obuf, gsem, wsem):
        wid = lax.axis_index("s") * NC + lax.axis_index("c")
        pltpu.sync_copy(x_hbm.at[pl.ds(wid * (128 * TT), 128 * TT)], xv)

        lane = lax.iota(jnp.int32, 16)
        lane200 = lane * TT

        def build_idx(t, s):
            for kk in range(8):
                v = plsc.load_gather(xv, [lane200 + (kk * 16 * TT + t)])
                widx[s, pl.ds(kk * 16, 16)] = lax.shift_right_logical(v, 1)
                hoff[s, pl.ds(kk * 16, 16)] = (v & 1) * 64

        def gather(s):
            return pltpu.make_async_copy(
                tw_hbm.at[widx.at[s]], rows.at[s], gsem.at[s]
            )

        def transpose(s):
            for kk in range(8):
                rowv = lane + kk * 16
                hv = hoff[s, pl.ds(kk * 16, 16)]
                for tc in range(8):
                    for ci in range(8):
                        v = plsc.load_gather(
                            rows.at[s], [rowv, hv + (tc * 8 + ci)]
                        )
                        obuf[s, tc, ci, pl.ds(kk * 16, 16)] = v

        def writeback(t, s):
            return pltpu.make_async_copy(
                obuf.at[s], out_hbm.at[t, :, wid], wsem.at[s]
            )

        build_idx(0, 0)
        gather(0).start()

        def body(t, carry):
            s = lax.rem(t, 2)
            sn = 1 - s

            @pl.when(t < TT - 1)
            def _():
                build_idx(t + 1, sn)

            gather(s).wait()

            @pl.when(t < TT - 1)
            def _():
                gather(sn).start()

            @pl.when(t >= 2)
            def _():
                writeback(t - 2, s).wait()

            transpose(s)
            writeback(t, s).start()
            return carry

        lax.fori_loop(0, TT, body, 0, unroll=False)
        writeback(TT - 2, 0).wait()
        writeback(TT - 1, 1).wait()

    return k(xr, tablew)


def kernel(x, table):
    xr = x.reshape(x.size).astype(jnp.int32)
    tablew = table.reshape(500000, 128)
    out5 = _fused_gather(xr, tablew)
    out = out5.transpose(2, 4, 0, 1, 3).reshape(4096, 200, 64)
    return out
